# Initial kernel scaffold; baseline (speedup 1.0000x reference)
import jax
import jax.numpy as jnp
from jax.experimental import pallas as pl


def kernel(x, edge_index, W1, att_src1, att_dst1, b1, W2, att_src2, att_dst2, b2):
    raise NotImplementedError("write your pallas kernel here")


# speedup vs baseline: 41.9466x; 41.9466x over previous
import functools

import jax
import jax.numpy as jnp
from jax import lax
from jax.experimental import pallas as pl
from jax.experimental.pallas import tpu as pltpu
from jax.experimental.pallas import tpu_sc as plsc

N = 10000
D = 128
E = 320000
H1, O1 = 8, 16

NPAD = 10240
RW = 144
TW = 16
NC, NS = 2, 16
NW = NC * NS
C = 128
ET = E + N
CPW = -(-ET // (NW * C))
EPAD = CPW * NW * C

BLK = 512
GRID = NPAD // BLK


def _prep1_body(x_ref, w_ref, as_ref, ad_ref, s_ref, t_ref):
    h = jnp.dot(x_ref[...], w_ref[...], preferred_element_type=jnp.float32)
    a_s = jnp.dot(h, as_ref[...], preferred_element_type=jnp.float32)
    a_d = jnp.dot(h, ad_ref[...], preferred_element_type=jnp.float32)
    z8 = jnp.zeros((BLK, 8), jnp.float32)
    s_ref[...] = jnp.concatenate([h, a_s, z8], axis=1)
    t_ref[...] = jnp.concatenate([a_d, z8], axis=1)


_prep1 = pl.pallas_call(
    _prep1_body,
    grid=(GRID,),
    in_specs=[
        pl.BlockSpec((BLK, D), lambda i: (i, 0)),
        pl.BlockSpec((D, D), lambda i: (0, 0)),
        pl.BlockSpec((D, H1), lambda i: (0, 0)),
        pl.BlockSpec((D, H1), lambda i: (0, 0)),
    ],
    out_specs=[
        pl.BlockSpec((BLK, RW), lambda i: (i, 0)),
        pl.BlockSpec((BLK, TW), lambda i: (i, 0)),
    ],
    out_shape=[
        jax.ShapeDtypeStruct((NPAD, RW), jnp.float32),
        jax.ShapeDtypeStruct((NPAD, TW), jnp.float32),
    ],
)


def _mid_body(acc_ref, b1_ref, w2_ref, rh_ref, vs_ref, vd_ref, s2_ref, t2_ref):
    a = acc_ref[0]
    b = acc_ref[1]
    out_un = a[:, :D] + b[:, :D]
    den = a[:, D:D + H1] + b[:, D:D + H1]
    den_rep = jnp.dot(den, rh_ref[...], preferred_element_type=jnp.float32)
    y = out_un / (den_rep + 1e-16) + b1_ref[...]
    x2 = jnp.where(y > 0.0, y, jnp.exp(y) - 1.0)
    h2 = jnp.dot(x2, w2_ref[...], preferred_element_type=jnp.float32)
    a2s = jnp.sum(h2 * vs_ref[...], axis=1, keepdims=True)
    a2d = jnp.sum(h2 * vd_ref[...], axis=1, keepdims=True)
    s2_ref[...] = jnp.concatenate(
        [h2, jnp.broadcast_to(a2s, (BLK, TW))], axis=1)
    t2_ref[...] = jnp.broadcast_to(a2d, (BLK, TW))


_mid = pl.pallas_call(
    _mid_body,
    grid=(GRID,),
    in_specs=[
        pl.BlockSpec((NC, BLK, RW), lambda i: (0, i, 0)),
        pl.BlockSpec((1, D), lambda i: (0, 0)),
        pl.BlockSpec((D, D), lambda i: (0, 0)),
        pl.BlockSpec((H1, D), lambda i: (0, 0)),
        pl.BlockSpec((1, D), lambda i: (0, 0)),
        pl.BlockSpec((1, D), lambda i: (0, 0)),
    ],
    out_specs=[
        pl.BlockSpec((BLK, RW), lambda i: (i, 0)),
        pl.BlockSpec((BLK, TW), lambda i: (i, 0)),
    ],
    out_shape=[
        jax.ShapeDtypeStruct((NPAD, RW), jnp.float32),
        jax.ShapeDtypeStruct((NPAD, TW), jnp.float32),
    ],
)


def _fin_body(acc_ref, b2_ref, o_ref):
    a = acc_ref[0]
    b = acc_ref[1]
    out_un = a[:, :D] + b[:, :D]
    den = a[:, D:D + 1] + b[:, D:D + 1]
    h = out_un / (den + 1e-16) + b2_ref[...]
    m = jnp.max(h, axis=1, keepdims=True)
    z = h - m
    o_ref[...] = z - jnp.log(jnp.sum(jnp.exp(z), axis=1, keepdims=True))


_fin = pl.pallas_call(
    _fin_body,
    grid=(GRID,),
    in_specs=[
        pl.BlockSpec((NC, BLK, RW), lambda i: (0, i, 0)),
        pl.BlockSpec((1, D), lambda i: (0, 0)),
    ],
    out_specs=pl.BlockSpec((BLK, D), lambda i: (i, 0)),
    out_shape=jax.ShapeDtypeStruct((NPAD, D), jnp.float32),
)

_GATHER_DNUMS = lax.GatherDimensionNumbers(
    offset_dims=(), collapsed_slice_dims=(0,), start_index_map=(0,))


def _make_edge_kernel(nheads):
    mesh = plsc.VectorSubcoreMesh(
        core_axis_name="c", subcore_axis_name="s",
        num_cores=NC, num_subcores=NS)
    rps = NPAD // NS
    zb = 16
    ob = 128

    @functools.partial(
        pl.kernel,
        out_type=jax.ShapeDtypeStruct((NC, NPAD, RW), jnp.float32),
        mesh=mesh,
        compiler_params=pltpu.CompilerParams(use_tc_tiling_on_sc=False),
        scratch_types=[
            pltpu.VMEM((C,), jnp.int32),
            pltpu.VMEM((C,), jnp.int32),
            pltpu.VMEM((C, RW), jnp.float32),
            pltpu.VMEM((C, TW), jnp.float32),
            pltpu.VMEM((zb, RW), jnp.float32),
            pltpu.VMEM_SHARED((NPAD, RW), jnp.float32),
            pltpu.SemaphoreType.DMA,
            pltpu.SemaphoreType.DMA,
        ],
    )
    def edge_kernel(s_hbm, t_hbm, src_hbm, dst_hbm, out_hbm,
                    idx_s, idx_d, rows_s, rows_d, zbuf, accum,
                    sem1, sem2):
        cid = lax.axis_index("c")
        sid = lax.axis_index("s")
        wid = cid * NS + sid

        for r in range(zb):
            for v in range(RW // 16):
                zbuf[r, pl.ds(v * 16, 16)] = jnp.zeros((16,), jnp.float32)
        row0 = sid * rps

        def zloop(i, carry):
            pltpu.sync_copy(zbuf, accum.at[pl.ds(row0 + i * zb, zb)])
            return carry

        lax.fori_loop(0, rps // zb, zloop, 0)
        plsc.subcore_barrier()

        base_e = wid * (CPW * C)

        def chunk(g, carry):
            b = base_e + g * C
            pltpu.sync_copy(src_hbm.at[pl.ds(b, C)], idx_s)
            pltpu.sync_copy(dst_hbm.at[pl.ds(b, C)], idx_d)
            cp1 = pltpu.async_copy(s_hbm.at[idx_s], rows_s, sem1)
            cp2 = pltpu.async_copy(t_hbm.at[idx_d], rows_d, sem2)
            cp1.wait()
            cp2.wait()

            def edge(e, ecarry):
                a = rows_s[e, pl.ds(D, 16)] + rows_d[e, :]
                s = jnp.exp(jnp.where(a >= 0.0, a, 0.2 * a))
                rows_s[e, pl.ds(D, 16)] = s
                for j in range(D // 16):
                    hj = rows_s[e, pl.ds(j * 16, 16)]
                    if nheads == 1:
                        sj = s
                    else:
                        sj = lax.gather(
                            s, jnp.full((16, 1), j, jnp.int32),
                            _GATHER_DNUMS, (1,),
                            mode=lax.GatherScatterMode.PROMISE_IN_BOUNDS)
                    rows_s[e, pl.ds(j * 16, 16)] = hj * sj
                return ecarry

            lax.fori_loop(0, C, edge, 0)
            pltpu.sync_copy(rows_s, accum.at[idx_d], add=True)
            return carry

        lax.fori_loop(0, CPW, chunk, 0)
        plsc.subcore_barrier()

        def oloop(i, carry):
            r = row0 + i * ob
            pltpu.sync_copy(accum.at[pl.ds(r, ob)],
                            out_hbm.at[cid, pl.ds(r, ob)])
            return carry

        lax.fori_loop(0, rps // ob, oloop, 0)

    return edge_kernel


_edge8 = _make_edge_kernel(H1)
_edge1 = _make_edge_kernel(1)


def kernel(x, edge_index, W1, att_src1, att_dst1, b1, W2, att_src2,
           att_dst2, b2):
    x = x.astype(jnp.float32)
    xp = jnp.zeros((NPAD, D), jnp.float32).at[:N].set(x)
    ei = edge_index.astype(jnp.int32)
    loop = jnp.arange(N, dtype=jnp.int32)
    pad = jnp.full((EPAD - ET,), NPAD - 1, jnp.int32)
    src = jnp.concatenate([ei[0], loop, pad])
    dst = jnp.concatenate([ei[1], loop, pad])

    eye8 = jnp.eye(H1, dtype=jnp.float32)
    a_s1 = (att_src1[:, :, None] * eye8[:, None, :]).reshape(D, H1)
    a_d1 = (att_dst1[:, :, None] * eye8[:, None, :]).reshape(D, H1)
    rh = (eye8[:, :, None] * jnp.ones((1, 1, O1), jnp.float32)).reshape(H1, D)

    s1, t1 = _prep1(xp, W1, a_s1, a_d1)
    acc1 = _edge8(s1, t1, src, dst)
    s2, t2 = _mid(acc1, b1.reshape(1, D), W2, rh,
                  att_src2.reshape(1, D), att_dst2.reshape(1, D))
    acc2 = _edge1(s2, t2, src, dst)
    out = _fin(acc2, b2.reshape(1, D))
    return out[:N]


# speedup vs baseline: 54.5108x; 1.2995x over previous
import functools

import jax
import jax.numpy as jnp
from jax import lax
from jax.experimental import pallas as pl
from jax.experimental.pallas import tpu as pltpu
from jax.experimental.pallas import tpu_sc as plsc

N = 10000
D = 128
E = 320000
H1, O1 = 8, 16

NPAD = 10240
RW = 144
TW = 16
NC, NS = 2, 16
NW = NC * NS
C = 64
ET = E + N
NB = 3
CPW = NB * (-(-ET // (NW * C * NB)))
EPAD = CPW * NW * C

BLK = 512
GRID = NPAD // BLK


def _prep1_body(x_ref, w_ref, as_ref, ad_ref, s_ref, t_ref):
    h = jnp.dot(x_ref[...], w_ref[...], preferred_element_type=jnp.float32)
    a_s = jnp.dot(h, as_ref[...], preferred_element_type=jnp.float32)
    a_d = jnp.dot(h, ad_ref[...], preferred_element_type=jnp.float32)
    z8 = jnp.zeros((BLK, 8), jnp.float32)
    s_ref[...] = jnp.concatenate([h, a_s, z8], axis=1)
    t_ref[...] = jnp.concatenate([a_d, z8], axis=1)


_prep1 = pl.pallas_call(
    _prep1_body,
    grid=(GRID,),
    in_specs=[
        pl.BlockSpec((BLK, D), lambda i: (i, 0)),
        pl.BlockSpec((D, D), lambda i: (0, 0)),
        pl.BlockSpec((D, H1), lambda i: (0, 0)),
        pl.BlockSpec((D, H1), lambda i: (0, 0)),
    ],
    out_specs=[
        pl.BlockSpec((BLK, RW), lambda i: (i, 0)),
        pl.BlockSpec((BLK, TW), lambda i: (i, 0)),
    ],
    out_shape=[
        jax.ShapeDtypeStruct((NPAD, RW), jnp.float32),
        jax.ShapeDtypeStruct((NPAD, TW), jnp.float32),
    ],
)


def _mid_body(acc_ref, b1_ref, w2_ref, rh_ref, vs_ref, vd_ref, s2_ref, t2_ref):
    a = acc_ref[0]
    b = acc_ref[1]
    out_un = a[:, :D] + b[:, :D]
    den = a[:, D:D + H1] + b[:, D:D + H1]
    den_rep = jnp.dot(den, rh_ref[...], preferred_element_type=jnp.float32)
    y = out_un / (den_rep + 1e-16) + b1_ref[...]
    x2 = jnp.where(y > 0.0, y, jnp.exp(y) - 1.0)
    h2 = jnp.dot(x2, w2_ref[...], preferred_element_type=jnp.float32)
    a2s = jnp.sum(h2 * vs_ref[...], axis=1, keepdims=True)
    a2d = jnp.sum(h2 * vd_ref[...], axis=1, keepdims=True)
    s2_ref[...] = jnp.concatenate(
        [h2, jnp.broadcast_to(a2s, (BLK, TW))], axis=1)
    t2_ref[...] = jnp.broadcast_to(a2d, (BLK, TW))


_mid = pl.pallas_call(
    _mid_body,
    grid=(GRID,),
    in_specs=[
        pl.BlockSpec((NC, BLK, RW), lambda i: (0, i, 0)),
        pl.BlockSpec((1, D), lambda i: (0, 0)),
        pl.BlockSpec((D, D), lambda i: (0, 0)),
        pl.BlockSpec((H1, D), lambda i: (0, 0)),
        pl.BlockSpec((1, D), lambda i: (0, 0)),
        pl.BlockSpec((1, D), lambda i: (0, 0)),
    ],
    out_specs=[
        pl.BlockSpec((BLK, RW), lambda i: (i, 0)),
        pl.BlockSpec((BLK, TW), lambda i: (i, 0)),
    ],
    out_shape=[
        jax.ShapeDtypeStruct((NPAD, RW), jnp.float32),
        jax.ShapeDtypeStruct((NPAD, TW), jnp.float32),
    ],
)


def _fin_body(acc_ref, b2_ref, o_ref):
    a = acc_ref[0]
    b = acc_ref[1]
    out_un = a[:, :D] + b[:, :D]
    den = a[:, D:D + 1] + b[:, D:D + 1]
    h = out_un / (den + 1e-16) + b2_ref[...]
    m = jnp.max(h, axis=1, keepdims=True)
    z = h - m
    o_ref[...] = z - jnp.log(jnp.sum(jnp.exp(z), axis=1, keepdims=True))


_fin = pl.pallas_call(
    _fin_body,
    grid=(GRID,),
    in_specs=[
        pl.BlockSpec((NC, BLK, RW), lambda i: (0, i, 0)),
        pl.BlockSpec((1, D), lambda i: (0, 0)),
    ],
    out_specs=pl.BlockSpec((BLK, D), lambda i: (i, 0)),
    out_shape=jax.ShapeDtypeStruct((NPAD, D), jnp.float32),
)

_GATHER_DNUMS = lax.GatherDimensionNumbers(
    offset_dims=(), collapsed_slice_dims=(0,), start_index_map=(0,))


def _make_edge_kernel(nheads):
    mesh = plsc.VectorSubcoreMesh(
        core_axis_name="c", subcore_axis_name="s",
        num_cores=NC, num_subcores=NS)
    rps = NPAD // NS
    zb = 16
    ob = 128

    @functools.partial(
        pl.kernel,
        out_type=jax.ShapeDtypeStruct((NC, NPAD, RW), jnp.float32),
        mesh=mesh,
        compiler_params=pltpu.CompilerParams(use_tc_tiling_on_sc=False),
        scratch_types=[
            pltpu.VMEM((NB, C), jnp.int32),
            pltpu.VMEM((NB, C), jnp.int32),
            pltpu.VMEM((NB, C, RW), jnp.float32),
            pltpu.VMEM((NB, C, TW), jnp.float32),
            pltpu.VMEM((zb, RW), jnp.float32),
            pltpu.VMEM_SHARED((NPAD, RW), jnp.float32),
            [pltpu.SemaphoreType.DMA] * NB,
            [pltpu.SemaphoreType.DMA] * NB,
        ],
    )
    def edge_kernel(s_hbm, t_hbm, src_hbm, dst_hbm, out_hbm,
                    idx_s, idx_d, rows_s, rows_d, zbuf, accum,
                    gsem, wsem):
        cid = lax.axis_index("c")
        sid = lax.axis_index("s")
        wid = cid * NS + sid

        for r in range(zb):
            for v in range(RW // 16):
                zbuf[r, pl.ds(v * 16, 16)] = jnp.zeros((16,), jnp.float32)
        row0 = sid * rps

        def zloop(i, carry):
            pltpu.sync_copy(zbuf, accum.at[pl.ds(row0 + i * zb, zb)])
            return carry

        lax.fori_loop(0, rps // zb, zloop, 0)
        plsc.subcore_barrier()

        base_e = wid * (CPW * C)

        def issue_gather(k, g):
            b = base_e + g * C
            pltpu.sync_copy(src_hbm.at[pl.ds(b, C)], idx_s.at[k])
            pltpu.sync_copy(dst_hbm.at[pl.ds(b, C)], idx_d.at[k])
            pltpu.async_copy(s_hbm.at[idx_s.at[k]], rows_s.at[k], gsem[k])
            pltpu.async_copy(t_hbm.at[idx_d.at[k]], rows_d.at[k], gsem[k])

        def wait_gather(k):
            pltpu.make_async_copy(
                s_hbm.at[idx_s.at[k]], rows_s.at[k], gsem[k]).wait()
            pltpu.make_async_copy(
                t_hbm.at[idx_d.at[k]], rows_d.at[k], gsem[k]).wait()

        def issue_scatter(k):
            pltpu.async_copy(rows_s.at[k], accum.at[idx_d.at[k]], wsem[k],
                             add=True)

        def wait_scatter(k):
            pltpu.make_async_copy(
                rows_s.at[k], accum.at[idx_d.at[k]], wsem[k]).wait()

        def compute(k):
            def edge(e, ecarry):
                a = rows_s[k, e, pl.ds(D, 16)] + rows_d[k, e, :]
                s = jnp.exp(jnp.where(a >= 0.0, a, 0.2 * a))
                rows_s[k, e, pl.ds(D, 16)] = s
                for j in range(D // 16):
                    hj = rows_s[k, e, pl.ds(j * 16, 16)]
                    if nheads == 1:
                        sj = s
                    else:
                        sj = lax.gather(
                            s, jnp.full((16, 1), j, jnp.int32),
                            _GATHER_DNUMS, (1,),
                            mode=lax.GatherScatterMode.PROMISE_IN_BOUNDS)
                    rows_s[k, e, pl.ds(j * 16, 16)] = hj * sj
                return ecarry

            lax.fori_loop(0, C, edge, 0)

        issue_gather(0, 0)
        issue_gather(1, 1)

        def trip(t, carry):
            g = t * NB
            for k in range(NB):
                gk = g + k
                wait_gather(k)
                compute(k)
                issue_scatter(k)
                kp = (k + 2) % NB
                if k == 0:
                    @pl.when(t > 0)
                    def _():
                        wait_scatter(kp)
                else:
                    wait_scatter(kp)

                @pl.when(gk + 2 < CPW)
                def _():
                    issue_gather(kp, gk + 2)
            return carry

        lax.fori_loop(0, CPW // NB, trip, 0)
        wait_scatter((CPW - 1) % NB)
        plsc.subcore_barrier()

        def oloop(i, carry):
            r = row0 + i * ob
            pltpu.sync_copy(accum.at[pl.ds(r, ob)],
                            out_hbm.at[cid, pl.ds(r, ob)])
            return carry

        lax.fori_loop(0, rps // ob, oloop, 0)

    return edge_kernel


_edge8 = _make_edge_kernel(H1)
_edge1 = _make_edge_kernel(1)


def kernel(x, edge_index, W1, att_src1, att_dst1, b1, W2, att_src2,
           att_dst2, b2):
    x = x.astype(jnp.float32)
    xp = jnp.zeros((NPAD, D), jnp.float32).at[:N].set(x)
    ei = edge_index.astype(jnp.int32)
    loop = jnp.arange(N, dtype=jnp.int32)
    pad = jnp.full((EPAD - ET,), NPAD - 1, jnp.int32)
    src = jnp.concatenate([ei[0], loop, pad])
    dst = jnp.concatenate([ei[1], loop, pad])

    eye8 = jnp.eye(H1, dtype=jnp.float32)
    a_s1 = (att_src1[:, :, None] * eye8[:, None, :]).reshape(D, H1)
    a_d1 = (att_dst1[:, :, None] * eye8[:, None, :]).reshape(D, H1)
    rh = (eye8[:, :, None] * jnp.ones((1, 1, O1), jnp.float32)).reshape(H1, D)

    s1, t1 = _prep1(xp, W1, a_s1, a_d1)
    acc1 = _edge8(s1, t1, src, dst)
    s2, t2 = _mid(acc1, b1.reshape(1, D), W2, rh,
                  att_src2.reshape(1, D), att_dst2.reshape(1, D))
    acc2 = _edge1(s2, t2, src, dst)
    out = _fin(acc2, b2.reshape(1, D))
    return out[:N]


# speedup vs baseline: 68.9554x; 1.2650x over previous
import functools

import jax
import jax.numpy as jnp
from jax import lax
from jax.experimental import pallas as pl
from jax.experimental.pallas import tpu as pltpu
from jax.experimental.pallas import tpu_sc as plsc

N = 10000
D = 128
E = 320000
H1, O1 = 8, 16

NPAD = 10240
RW = 144
TW = 16
NC, NS = 2, 16
NW = NC * NS
C = 64
ET = E + N
NB = 3
CPW = NB * (-(-ET // (NW * C * NB)))
EPAD = CPW * NW * C

BLK = 512
GRID = NPAD // BLK


def _prep1_body(x_ref, w_ref, as_ref, ad_ref, s_ref, t_ref):
    h = jnp.dot(x_ref[...], w_ref[...], preferred_element_type=jnp.float32)
    a_s = jnp.dot(h, as_ref[...], preferred_element_type=jnp.float32)
    a_d = jnp.dot(h, ad_ref[...], preferred_element_type=jnp.float32)
    z8 = jnp.zeros((BLK, 8), jnp.float32)
    s_ref[...] = jnp.concatenate([h, a_s, z8], axis=1)
    t_ref[...] = jnp.concatenate([a_d, z8], axis=1)


_prep1 = pl.pallas_call(
    _prep1_body,
    grid=(GRID,),
    in_specs=[
        pl.BlockSpec((BLK, D), lambda i: (i, 0)),
        pl.BlockSpec((D, D), lambda i: (0, 0)),
        pl.BlockSpec((D, H1), lambda i: (0, 0)),
        pl.BlockSpec((D, H1), lambda i: (0, 0)),
    ],
    out_specs=[
        pl.BlockSpec((BLK, RW), lambda i: (i, 0)),
        pl.BlockSpec((BLK, TW), lambda i: (i, 0)),
    ],
    out_shape=[
        jax.ShapeDtypeStruct((NPAD, RW), jnp.float32),
        jax.ShapeDtypeStruct((NPAD, TW), jnp.float32),
    ],
)


def _mid_body(acc_ref, b1_ref, w2_ref, rh_ref, vs_ref, vd_ref, s2_ref, t2_ref):
    a = acc_ref[0]
    b = acc_ref[1]
    out_un = a[:, :D] + b[:, :D]
    den = a[:, D:D + H1] + b[:, D:D + H1]
    den_rep = jnp.dot(den, rh_ref[...], preferred_element_type=jnp.float32)
    y = out_un / (den_rep + 1e-16) + b1_ref[...]
    x2 = jnp.where(y > 0.0, y, jnp.exp(y) - 1.0)
    h2 = jnp.dot(x2, w2_ref[...], preferred_element_type=jnp.float32)
    a2s = jnp.sum(h2 * vs_ref[...], axis=1, keepdims=True)
    a2d = jnp.sum(h2 * vd_ref[...], axis=1, keepdims=True)
    s2_ref[...] = jnp.concatenate(
        [h2, jnp.broadcast_to(a2s, (BLK, TW))], axis=1)
    t2_ref[...] = jnp.broadcast_to(a2d, (BLK, TW))


_mid = pl.pallas_call(
    _mid_body,
    grid=(GRID,),
    in_specs=[
        pl.BlockSpec((NC, BLK, RW), lambda i: (0, i, 0)),
        pl.BlockSpec((1, D), lambda i: (0, 0)),
        pl.BlockSpec((D, D), lambda i: (0, 0)),
        pl.BlockSpec((H1, D), lambda i: (0, 0)),
        pl.BlockSpec((1, D), lambda i: (0, 0)),
        pl.BlockSpec((1, D), lambda i: (0, 0)),
    ],
    out_specs=[
        pl.BlockSpec((BLK, RW), lambda i: (i, 0)),
        pl.BlockSpec((BLK, TW), lambda i: (i, 0)),
    ],
    out_shape=[
        jax.ShapeDtypeStruct((NPAD, RW), jnp.float32),
        jax.ShapeDtypeStruct((NPAD, TW), jnp.float32),
    ],
)


def _fin_body(acc_ref, b2_ref, o_ref):
    a = acc_ref[0]
    b = acc_ref[1]
    out_un = a[:, :D] + b[:, :D]
    den = a[:, D:D + 1] + b[:, D:D + 1]
    h = out_un / (den + 1e-16) + b2_ref[...]
    m = jnp.max(h, axis=1, keepdims=True)
    z = h - m
    o_ref[...] = z - jnp.log(jnp.sum(jnp.exp(z), axis=1, keepdims=True))


_fin = pl.pallas_call(
    _fin_body,
    grid=(GRID,),
    in_specs=[
        pl.BlockSpec((NC, BLK, RW), lambda i: (0, i, 0)),
        pl.BlockSpec((1, D), lambda i: (0, 0)),
    ],
    out_specs=pl.BlockSpec((BLK, D), lambda i: (i, 0)),
    out_shape=jax.ShapeDtypeStruct((NPAD, D), jnp.float32),
)

_GATHER_DNUMS = lax.GatherDimensionNumbers(
    offset_dims=(), collapsed_slice_dims=(0,), start_index_map=(0,))


def _make_edge_kernel(nheads):
    mesh = plsc.VectorSubcoreMesh(
        core_axis_name="c", subcore_axis_name="s",
        num_cores=NC, num_subcores=NS)
    rps = NPAD // NS
    zb = 16
    ob = 128

    @functools.partial(
        pl.kernel,
        out_type=jax.ShapeDtypeStruct((NC, NPAD, RW), jnp.float32),
        mesh=mesh,
        compiler_params=pltpu.CompilerParams(use_tc_tiling_on_sc=False),
        scratch_types=[
            pltpu.VMEM((NB, C), jnp.int32),
            pltpu.VMEM((NB, C), jnp.int32),
            pltpu.VMEM((NB, C, RW), jnp.float32),
            pltpu.VMEM((NB, C, TW), jnp.float32),
            pltpu.VMEM((zb, RW), jnp.float32),
            pltpu.VMEM_SHARED((NPAD, RW), jnp.float32),
            [pltpu.SemaphoreType.DMA] * NB,
            [pltpu.SemaphoreType.DMA] * NB,
        ],
    )
    def edge_kernel(s_hbm, t_hbm, src_hbm, dst_hbm, out_hbm,
                    idx_s, idx_d, rows_s, rows_d, zbuf, accum,
                    gsem, wsem):
        cid = lax.axis_index("c")
        sid = lax.axis_index("s")
        wid = cid * NS + sid

        for r in range(zb):
            for v in range(RW // 16):
                zbuf[r, pl.ds(v * 16, 16)] = jnp.zeros((16,), jnp.float32)
        row0 = sid * rps

        def zloop(i, carry):
            pltpu.sync_copy(zbuf, accum.at[pl.ds(row0 + i * zb, zb)])
            return carry

        lax.fori_loop(0, rps // zb, zloop, 0)
        plsc.subcore_barrier()

        base_e = wid * (CPW * C)

        def issue_gather(k, g):
            b = base_e + g * C
            pltpu.sync_copy(src_hbm.at[pl.ds(b, C)], idx_s.at[k])
            pltpu.sync_copy(dst_hbm.at[pl.ds(b, C)], idx_d.at[k])
            pltpu.async_copy(s_hbm.at[idx_s.at[k]], rows_s.at[k], gsem[k])
            pltpu.async_copy(t_hbm.at[idx_d.at[k]], rows_d.at[k], gsem[k])

        def wait_gather(k):
            pltpu.make_async_copy(
                s_hbm.at[idx_s.at[k]], rows_s.at[k], gsem[k]).wait()
            pltpu.make_async_copy(
                t_hbm.at[idx_d.at[k]], rows_d.at[k], gsem[k]).wait()

        def issue_scatter(k):
            pltpu.async_copy(rows_s.at[k], accum.at[idx_d.at[k]], wsem[k],
                             add=True)

        def wait_scatter(k):
            pltpu.make_async_copy(
                rows_s.at[k], accum.at[idx_d.at[k]], wsem[k]).wait()

        def compute(k):
            @plsc.parallel_loop(0, C, 1, unroll=4)
            def edge(e):
                a = rows_s[k, e, pl.ds(D, 16)] + rows_d[k, e, :]
                s = jnp.exp(jnp.where(a >= 0.0, a, 0.2 * a))
                rows_s[k, e, pl.ds(D, 16)] = s
                for j in range(D // 16):
                    hj = rows_s[k, e, pl.ds(j * 16, 16)]
                    if nheads == 1:
                        sj = s
                    else:
                        sj = lax.gather(
                            s, jnp.full((16, 1), j, jnp.int32),
                            _GATHER_DNUMS, (1,),
                            mode=lax.GatherScatterMode.PROMISE_IN_BOUNDS)
                    rows_s[k, e, pl.ds(j * 16, 16)] = hj * sj

        issue_gather(0, 0)
        issue_gather(1, 1)

        def trip(t, carry):
            g = t * NB
            for k in range(NB):
                gk = g + k
                wait_gather(k)
                compute(k)
                issue_scatter(k)
                kp = (k + 2) % NB
                if k == 0:
                    @pl.when(t > 0)
                    def _():
                        wait_scatter(kp)
                else:
                    wait_scatter(kp)

                @pl.when(gk + 2 < CPW)
                def _():
                    issue_gather(kp, gk + 2)
            return carry

        lax.fori_loop(0, CPW // NB, trip, 0)
        wait_scatter((CPW - 1) % NB)
        plsc.subcore_barrier()

        def oloop(i, carry):
            r = row0 + i * ob
            pltpu.sync_copy(accum.at[pl.ds(r, ob)],
                            out_hbm.at[cid, pl.ds(r, ob)])
            return carry

        lax.fori_loop(0, rps // ob, oloop, 0)

    return edge_kernel


_edge8 = _make_edge_kernel(H1)
_edge1 = _make_edge_kernel(1)


def kernel(x, edge_index, W1, att_src1, att_dst1, b1, W2, att_src2,
           att_dst2, b2):
    x = x.astype(jnp.float32)
    xp = jnp.zeros((NPAD, D), jnp.float32).at[:N].set(x)
    ei = edge_index.astype(jnp.int32)
    loop = jnp.arange(N, dtype=jnp.int32)
    pad = jnp.full((EPAD - ET,), NPAD - 1, jnp.int32)
    src = jnp.concatenate([ei[0], loop, pad])
    dst = jnp.concatenate([ei[1], loop, pad])

    eye8 = jnp.eye(H1, dtype=jnp.float32)
    a_s1 = (att_src1[:, :, None] * eye8[:, None, :]).reshape(D, H1)
    a_d1 = (att_dst1[:, :, None] * eye8[:, None, :]).reshape(D, H1)
    rh = (eye8[:, :, None] * jnp.ones((1, 1, O1), jnp.float32)).reshape(H1, D)

    s1, t1 = _prep1(xp, W1, a_s1, a_d1)
    acc1 = _edge8(s1, t1, src, dst)
    s2, t2 = _mid(acc1, b1.reshape(1, D), W2, rh,
                  att_src2.reshape(1, D), att_dst2.reshape(1, D))
    acc2 = _edge1(s2, t2, src, dst)
    out = _fin(acc2, b2.reshape(1, D))
    return out[:N]


# speedup vs baseline: 71.6668x; 1.0393x over previous
import functools

import numpy as np

import jax
import jax.numpy as jnp
from jax import lax
from jax.experimental import pallas as pl
from jax.experimental.pallas import tpu as pltpu
from jax.experimental.pallas import tpu_sc as plsc

N = 10000
D = 128
E = 320000
H1, O1 = 8, 16

NPAD = 10240
RW = 144
SW = 80
TW = 16
NC, NS = 2, 16
NW = NC * NS
C = 48
ET = E + N
NB = 3
CPW = NB * (-(-ET // (NW * C * NB)))
EPAD = CPW * NW * C

BLK = 512
GRID = NPAD // BLK

_E_NP = np.zeros((D, D // 2), np.float32)
_O_NP = np.zeros((D, D // 2), np.float32)
for _j in range(D // 2):
    _E_NP[2 * _j, _j] = 1.0
    _O_NP[2 * _j + 1, _j] = 1.0
_P_NP = np.zeros((D, D), np.float32)
for _g in range(4):
    for _par in range(2):
        for _i in range(16):
            _P_NP[32 * _g + 16 * _par + _i, 32 * _g + 2 * _i + _par] = 1.0


def _prep1_body(x_ref, w_ref, as_ref, ad_ref, em_ref, om_ref, s_ref, t_ref):
    h = jnp.dot(x_ref[...], w_ref[...], preferred_element_type=jnp.float32)
    a_s = jnp.dot(h, as_ref[...], preferred_element_type=jnp.float32)
    a_d = jnp.dot(h, ad_ref[...], preferred_element_type=jnp.float32)
    he = jnp.dot(h, em_ref[...], preferred_element_type=jnp.float32)
    ho = jnp.dot(h, om_ref[...], preferred_element_type=jnp.float32)
    ue = lax.bitcast_convert_type(
        he.astype(jnp.bfloat16), jnp.uint16).astype(jnp.uint32)
    uo = lax.bitcast_convert_type(
        ho.astype(jnp.bfloat16), jnp.uint16).astype(jnp.uint32)
    packed = lax.bitcast_convert_type(ue | (uo << 16), jnp.float32)
    z8 = jnp.zeros((BLK, 8), jnp.float32)
    s_ref[...] = jnp.concatenate([packed, a_s, z8], axis=1)
    t_ref[...] = jnp.concatenate([a_d, z8], axis=1)


_prep1 = pl.pallas_call(
    _prep1_body,
    grid=(GRID,),
    in_specs=[
        pl.BlockSpec((BLK, D), lambda i: (i, 0)),
        pl.BlockSpec((D, D), lambda i: (0, 0)),
        pl.BlockSpec((D, H1), lambda i: (0, 0)),
        pl.BlockSpec((D, H1), lambda i: (0, 0)),
        pl.BlockSpec((D, D // 2), lambda i: (0, 0)),
        pl.BlockSpec((D, D // 2), lambda i: (0, 0)),
    ],
    out_specs=[
        pl.BlockSpec((BLK, SW), lambda i: (i, 0)),
        pl.BlockSpec((BLK, TW), lambda i: (i, 0)),
    ],
    out_shape=[
        jax.ShapeDtypeStruct((NPAD, SW), jnp.float32),
        jax.ShapeDtypeStruct((NPAD, TW), jnp.float32),
    ],
)


def _mid_body(acc_ref, b1_ref, w2_ref, rh_ref, p_ref, em_ref, om_ref,
              vs_ref, vd_ref, s2_ref, t2_ref):
    a = acc_ref[0]
    b = acc_ref[1]
    out_un = jnp.dot(a[:, :D] + b[:, :D], p_ref[...],
                     preferred_element_type=jnp.float32)
    den = a[:, D:D + H1] + b[:, D:D + H1]
    den_rep = jnp.dot(den, rh_ref[...], preferred_element_type=jnp.float32)
    y = out_un / (den_rep + 1e-16) + b1_ref[...]
    x2 = jnp.where(y > 0.0, y, jnp.exp(y) - 1.0)
    h2 = jnp.dot(x2, w2_ref[...], preferred_element_type=jnp.float32)
    a2s = jnp.sum(h2 * vs_ref[...], axis=1, keepdims=True)
    a2d = jnp.sum(h2 * vd_ref[...], axis=1, keepdims=True)
    he = jnp.dot(h2, em_ref[...], preferred_element_type=jnp.float32)
    ho = jnp.dot(h2, om_ref[...], preferred_element_type=jnp.float32)
    ue = lax.bitcast_convert_type(
        he.astype(jnp.bfloat16), jnp.uint16).astype(jnp.uint32)
    uo = lax.bitcast_convert_type(
        ho.astype(jnp.bfloat16), jnp.uint16).astype(jnp.uint32)
    packed = lax.bitcast_convert_type(ue | (uo << 16), jnp.float32)
    s2_ref[...] = jnp.concatenate(
        [packed, jnp.broadcast_to(a2s, (BLK, TW))], axis=1)
    t2_ref[...] = jnp.broadcast_to(a2d, (BLK, TW))


_mid = pl.pallas_call(
    _mid_body,
    grid=(GRID,),
    in_specs=[
        pl.BlockSpec((NC, BLK, RW), lambda i: (0, i, 0)),
        pl.BlockSpec((1, D), lambda i: (0, 0)),
        pl.BlockSpec((D, D), lambda i: (0, 0)),
        pl.BlockSpec((H1, D), lambda i: (0, 0)),
        pl.BlockSpec((D, D), lambda i: (0, 0)),
        pl.BlockSpec((D, D // 2), lambda i: (0, 0)),
        pl.BlockSpec((D, D // 2), lambda i: (0, 0)),
        pl.BlockSpec((1, D), lambda i: (0, 0)),
        pl.BlockSpec((1, D), lambda i: (0, 0)),
    ],
    out_specs=[
        pl.BlockSpec((BLK, SW), lambda i: (i, 0)),
        pl.BlockSpec((BLK, TW), lambda i: (i, 0)),
    ],
    out_shape=[
        jax.ShapeDtypeStruct((NPAD, SW), jnp.float32),
        jax.ShapeDtypeStruct((NPAD, TW), jnp.float32),
    ],
)


def _fin_body(acc_ref, b2_ref, p_ref, o_ref):
    a = acc_ref[0]
    b = acc_ref[1]
    out_un = jnp.dot(a[:, :D] + b[:, :D], p_ref[...],
                     preferred_element_type=jnp.float32)
    den = a[:, D:D + 1] + b[:, D:D + 1]
    h = out_un / (den + 1e-16) + b2_ref[...]
    m = jnp.max(h, axis=1, keepdims=True)
    z = h - m
    o_ref[...] = z - jnp.log(jnp.sum(jnp.exp(z), axis=1, keepdims=True))


_fin = pl.pallas_call(
    _fin_body,
    grid=(GRID,),
    in_specs=[
        pl.BlockSpec((NC, BLK, RW), lambda i: (0, i, 0)),
        pl.BlockSpec((1, D), lambda i: (0, 0)),
        pl.BlockSpec((D, D), lambda i: (0, 0)),
    ],
    out_specs=pl.BlockSpec((BLK, D), lambda i: (i, 0)),
    out_shape=jax.ShapeDtypeStruct((NPAD, D), jnp.float32),
)


_GATHER_DNUMS = lax.GatherDimensionNumbers(
    offset_dims=(), collapsed_slice_dims=(0,), start_index_map=(0,))


def _make_edge_kernel(nheads):
    mesh = plsc.VectorSubcoreMesh(
        core_axis_name="c", subcore_axis_name="s",
        num_cores=NC, num_subcores=NS)
    rps = NPAD // NS
    zb = 16
    ob = 128

    @functools.partial(
        pl.kernel,
        out_type=jax.ShapeDtypeStruct((NC, NPAD, RW), jnp.float32),
        mesh=mesh,
        compiler_params=pltpu.CompilerParams(use_tc_tiling_on_sc=False),
        scratch_types=[
            pltpu.VMEM((NB, C), jnp.int32),
            pltpu.VMEM((NB, C), jnp.int32),
            pltpu.VMEM((NB, C, SW), jnp.float32),
            pltpu.VMEM((NB, C, TW), jnp.float32),
            pltpu.VMEM((NB, C, RW), jnp.float32),
            pltpu.VMEM((zb, RW), jnp.float32),
            pltpu.VMEM_SHARED((NPAD, RW), jnp.float32),
            [pltpu.SemaphoreType.DMA] * NB,
            [pltpu.SemaphoreType.DMA] * NB,
        ],
    )
    def edge_kernel(s_hbm, t_hbm, src_hbm, dst_hbm, out_hbm,
                    idx_s, idx_d, rows_s, rows_d, wbuf, zbuf, accum,
                    gsem, wsem):
        cid = lax.axis_index("c")
        sid = lax.axis_index("s")
        wid = cid * NS + sid

        for r in range(zb):
            for v in range(RW // 16):
                zbuf[r, pl.ds(v * 16, 16)] = jnp.zeros((16,), jnp.float32)
        row0 = sid * rps

        def zloop(i, carry):
            pltpu.sync_copy(zbuf, accum.at[pl.ds(row0 + i * zb, zb)])
            return carry

        lax.fori_loop(0, rps // zb, zloop, 0)
        plsc.subcore_barrier()

        base_e = wid * (CPW * C)

        def issue_gather(k, g):
            b = base_e + g * C
            pltpu.sync_copy(src_hbm.at[pl.ds(b, C)], idx_s.at[k])
            pltpu.sync_copy(dst_hbm.at[pl.ds(b, C)], idx_d.at[k])
            pltpu.async_copy(s_hbm.at[idx_s.at[k]], rows_s.at[k], gsem[k])
            pltpu.async_copy(t_hbm.at[idx_d.at[k]], rows_d.at[k], gsem[k])

        def wait_gather(k):
            pltpu.make_async_copy(
                s_hbm.at[idx_s.at[k]], rows_s.at[k], gsem[k]).wait()
            pltpu.make_async_copy(
                t_hbm.at[idx_d.at[k]], rows_d.at[k], gsem[k]).wait()

        def issue_scatter(k):
            pltpu.async_copy(wbuf.at[k], accum.at[idx_d.at[k]], wsem[k],
                             add=True)

        def wait_scatter(k):
            pltpu.make_async_copy(
                wbuf.at[k], accum.at[idx_d.at[k]], wsem[k]).wait()

        def compute(k):
            lane_lo = lax.iota(jnp.int32, 16) < 8

            @plsc.parallel_loop(0, C, 1, unroll=4)
            def edge(e):
                a = rows_s[k, e, pl.ds(D // 2, 16)] + rows_d[k, e, :]
                s = jnp.exp(jnp.where(a >= 0.0, a, 0.2 * a))
                wbuf[k, e, pl.ds(D, 16)] = s
                for g4 in range(4):
                    u = lax.bitcast_convert_type(
                        rows_s[k, e, pl.ds(16 * g4, 16)], jnp.int32)
                    he = lax.bitcast_convert_type(u << 16, jnp.float32)
                    ho = lax.bitcast_convert_type(
                        u & jnp.int32(-65536), jnp.float32)
                    if nheads == 1:
                        sg = s
                    else:
                        s_lo = lax.gather(
                            s, jnp.full((16, 1), 2 * g4, jnp.int32),
                            _GATHER_DNUMS, (1,),
                            mode=lax.GatherScatterMode.PROMISE_IN_BOUNDS)
                        s_hi = lax.gather(
                            s, jnp.full((16, 1), 2 * g4 + 1, jnp.int32),
                            _GATHER_DNUMS, (1,),
                            mode=lax.GatherScatterMode.PROMISE_IN_BOUNDS)
                        sg = jnp.where(lane_lo, s_lo, s_hi)
                    wbuf[k, e, pl.ds(32 * g4, 16)] = he * sg
                    wbuf[k, e, pl.ds(32 * g4 + 16, 16)] = ho * sg

        issue_gather(0, 0)
        issue_gather(1, 1)

        def trip(t, carry):
            g = t * NB
            for k in range(NB):
                gk = g + k
                wait_gather(k)
                compute(k)
                issue_scatter(k)
                kp = (k + 2) % NB
                if k == 0:
                    @pl.when(t > 0)
                    def _():
                        wait_scatter(kp)
                else:
                    wait_scatter(kp)

                @pl.when(gk + 2 < CPW)
                def _():
                    issue_gather(kp, gk + 2)
            return carry

        lax.fori_loop(0, CPW // NB, trip, 0)
        wait_scatter((CPW - 1) % NB)
        plsc.subcore_barrier()

        def oloop(i, carry):
            r = row0 + i * ob
            pltpu.sync_copy(accum.at[pl.ds(r, ob)],
                            out_hbm.at[cid, pl.ds(r, ob)])
            return carry

        lax.fori_loop(0, rps // ob, oloop, 0)

    return edge_kernel


_edge8 = _make_edge_kernel(H1)
_edge1 = _make_edge_kernel(1)


def kernel(x, edge_index, W1, att_src1, att_dst1, b1, W2, att_src2,
           att_dst2, b2):
    x = x.astype(jnp.float32)
    xp = jnp.zeros((NPAD, D), jnp.float32).at[:N].set(x)
    ei = edge_index.astype(jnp.int32)
    loop = jnp.arange(N, dtype=jnp.int32)
    pad = jnp.full((EPAD - ET,), NPAD - 1, jnp.int32)
    src = jnp.concatenate([ei[0], loop, pad])
    dst = jnp.concatenate([ei[1], loop, pad])

    eye8 = jnp.eye(H1, dtype=jnp.float32)
    a_s1 = (att_src1[:, :, None] * eye8[:, None, :]).reshape(D, H1)
    a_d1 = (att_dst1[:, :, None] * eye8[:, None, :]).reshape(D, H1)
    rh = (eye8[:, :, None] * jnp.ones((1, 1, O1), jnp.float32)).reshape(H1, D)
    emat = jnp.asarray(_E_NP)
    omat = jnp.asarray(_O_NP)
    pmat = jnp.asarray(_P_NP)

    s1, t1 = _prep1(xp, W1, a_s1, a_d1, emat, omat)
    acc1 = _edge8(s1, t1, src, dst)
    s2, t2 = _mid(acc1, b1.reshape(1, D), W2, rh, pmat, emat, omat,
                  att_src2.reshape(1, D), att_dst2.reshape(1, D))
    acc2 = _edge1(s2, t2, src, dst)
    out = _fin(acc2, b2.reshape(1, D), pmat)
    return out[:N]


# speedup vs baseline: 71.7795x; 1.0016x over previous
import functools

import numpy as np

import jax
import jax.numpy as jnp
from jax import lax
from jax.experimental import pallas as pl
from jax.experimental.pallas import tpu as pltpu
from jax.experimental.pallas import tpu_sc as plsc

N = 10000
D = 128
E = 320000
H1, O1 = 8, 16

NPAD = 10240
RW = 144
SW = 80
TW = 16
NC, NS = 2, 16
NW = NC * NS
C = 48
ET = E + N
NB = 3
CPW = NB * (-(-ET // (NW * C * NB)))
EPAD = CPW * NW * C

BLK = 512
GRID = NPAD // BLK

_E_NP = np.zeros((D, D // 2), np.float32)
_O_NP = np.zeros((D, D // 2), np.float32)
for _j in range(D // 2):
    _E_NP[2 * _j, _j] = 1.0
    _O_NP[2 * _j + 1, _j] = 1.0
_P_NP = np.zeros((D, D), np.float32)
for _g in range(4):
    for _par in range(2):
        for _i in range(16):
            _P_NP[32 * _g + 16 * _par + _i, 32 * _g + 2 * _i + _par] = 1.0


def _prep1_body(x_ref, w_ref, as_ref, ad_ref, em_ref, om_ref, s_ref, t_ref):
    h = jnp.dot(x_ref[...], w_ref[...], preferred_element_type=jnp.float32)
    a_s = jnp.dot(h, as_ref[...], preferred_element_type=jnp.float32)
    a_d = jnp.dot(h, ad_ref[...], preferred_element_type=jnp.float32)
    he = jnp.dot(h, em_ref[...], preferred_element_type=jnp.float32)
    ho = jnp.dot(h, om_ref[...], preferred_element_type=jnp.float32)
    ue = lax.bitcast_convert_type(
        he.astype(jnp.bfloat16), jnp.uint16).astype(jnp.uint32)
    uo = lax.bitcast_convert_type(
        ho.astype(jnp.bfloat16), jnp.uint16).astype(jnp.uint32)
    packed = lax.bitcast_convert_type(ue | (uo << 16), jnp.float32)
    z8 = jnp.zeros((BLK, 8), jnp.float32)
    s_ref[...] = jnp.concatenate([packed, a_s, z8], axis=1)
    t_ref[...] = jnp.concatenate([a_d, z8], axis=1)


_prep1 = pl.pallas_call(
    _prep1_body,
    grid=(GRID,),
    in_specs=[
        pl.BlockSpec((BLK, D), lambda i: (i, 0)),
        pl.BlockSpec((D, D), lambda i: (0, 0)),
        pl.BlockSpec((D, H1), lambda i: (0, 0)),
        pl.BlockSpec((D, H1), lambda i: (0, 0)),
        pl.BlockSpec((D, D // 2), lambda i: (0, 0)),
        pl.BlockSpec((D, D // 2), lambda i: (0, 0)),
    ],
    out_specs=[
        pl.BlockSpec((BLK, SW), lambda i: (i, 0)),
        pl.BlockSpec((BLK, TW), lambda i: (i, 0)),
    ],
    out_shape=[
        jax.ShapeDtypeStruct((NPAD, SW), jnp.float32),
        jax.ShapeDtypeStruct((NPAD, TW), jnp.float32),
    ],
)


def _mid_body(acc_ref, b1_ref, w2_ref, rh_ref, p_ref, em_ref, om_ref,
              vs_ref, vd_ref, s2_ref, t2_ref):
    a = acc_ref[0]
    b = acc_ref[1]
    out_un = jnp.dot(a[:, :D] + b[:, :D], p_ref[...],
                     preferred_element_type=jnp.float32)
    den = a[:, D:D + H1] + b[:, D:D + H1]
    den_rep = jnp.dot(den, rh_ref[...], preferred_element_type=jnp.float32)
    y = out_un / (den_rep + 1e-16) + b1_ref[...]
    x2 = jnp.where(y > 0.0, y, jnp.exp(y) - 1.0)
    h2 = jnp.dot(x2, w2_ref[...], preferred_element_type=jnp.float32)
    a2s = jnp.sum(h2 * vs_ref[...], axis=1, keepdims=True)
    a2d = jnp.sum(h2 * vd_ref[...], axis=1, keepdims=True)
    he = jnp.dot(h2, em_ref[...], preferred_element_type=jnp.float32)
    ho = jnp.dot(h2, om_ref[...], preferred_element_type=jnp.float32)
    ue = lax.bitcast_convert_type(
        he.astype(jnp.bfloat16), jnp.uint16).astype(jnp.uint32)
    uo = lax.bitcast_convert_type(
        ho.astype(jnp.bfloat16), jnp.uint16).astype(jnp.uint32)
    packed = lax.bitcast_convert_type(ue | (uo << 16), jnp.float32)
    s2_ref[...] = jnp.concatenate(
        [packed, jnp.broadcast_to(a2s, (BLK, TW))], axis=1)
    t2_ref[...] = jnp.broadcast_to(a2d, (BLK, TW))


_mid = pl.pallas_call(
    _mid_body,
    grid=(GRID,),
    in_specs=[
        pl.BlockSpec((NC, BLK, RW), lambda i: (0, i, 0)),
        pl.BlockSpec((1, D), lambda i: (0, 0)),
        pl.BlockSpec((D, D), lambda i: (0, 0)),
        pl.BlockSpec((H1, D), lambda i: (0, 0)),
        pl.BlockSpec((D, D), lambda i: (0, 0)),
        pl.BlockSpec((D, D // 2), lambda i: (0, 0)),
        pl.BlockSpec((D, D // 2), lambda i: (0, 0)),
        pl.BlockSpec((1, D), lambda i: (0, 0)),
        pl.BlockSpec((1, D), lambda i: (0, 0)),
    ],
    out_specs=[
        pl.BlockSpec((BLK, SW), lambda i: (i, 0)),
        pl.BlockSpec((BLK, TW), lambda i: (i, 0)),
    ],
    out_shape=[
        jax.ShapeDtypeStruct((NPAD, SW), jnp.float32),
        jax.ShapeDtypeStruct((NPAD, TW), jnp.float32),
    ],
)


def _fin_body(acc_ref, b2_ref, p_ref, o_ref):
    a = acc_ref[0]
    b = acc_ref[1]
    out_un = jnp.dot(a[:, :D] + b[:, :D], p_ref[...],
                     preferred_element_type=jnp.float32)
    den = a[:, D:D + 1] + b[:, D:D + 1]
    h = out_un / (den + 1e-16) + b2_ref[...]
    m = jnp.max(h, axis=1, keepdims=True)
    z = h - m
    o_ref[...] = z - jnp.log(jnp.sum(jnp.exp(z), axis=1, keepdims=True))


_fin = pl.pallas_call(
    _fin_body,
    grid=(GRID,),
    in_specs=[
        pl.BlockSpec((NC, BLK, RW), lambda i: (0, i, 0)),
        pl.BlockSpec((1, D), lambda i: (0, 0)),
        pl.BlockSpec((D, D), lambda i: (0, 0)),
    ],
    out_specs=pl.BlockSpec((BLK, D), lambda i: (i, 0)),
    out_shape=jax.ShapeDtypeStruct((NPAD, D), jnp.float32),
)


_GATHER_DNUMS = lax.GatherDimensionNumbers(
    offset_dims=(), collapsed_slice_dims=(0,), start_index_map=(0,))


def _make_edge_kernel(nheads):
    mesh = plsc.VectorSubcoreMesh(
        core_axis_name="c", subcore_axis_name="s",
        num_cores=NC, num_subcores=NS)
    rps = NPAD // NS
    zb = 16
    ob = 128

    @functools.partial(
        pl.kernel,
        out_type=jax.ShapeDtypeStruct((NC, NPAD, RW), jnp.float32),
        mesh=mesh,
        compiler_params=pltpu.CompilerParams(use_tc_tiling_on_sc=False),
        scratch_types=[
            pltpu.VMEM((NB, C), jnp.int32),
            pltpu.VMEM((NB, C), jnp.int32),
            pltpu.VMEM((NB, C, SW), jnp.float32),
            pltpu.VMEM((NB, C, TW), jnp.float32),
            pltpu.VMEM((NB, C, RW), jnp.float32),
            pltpu.VMEM((zb, RW), jnp.float32),
            pltpu.VMEM_SHARED((NPAD, RW), jnp.float32),
            [pltpu.SemaphoreType.DMA] * NB,
            [pltpu.SemaphoreType.DMA] * NB,
        ],
    )
    def edge_kernel(s_hbm, t_hbm, src_hbm, dst_hbm, out_hbm,
                    idx_s, idx_d, rows_s, rows_d, wbuf, zbuf, accum,
                    gsem, wsem):
        cid = lax.axis_index("c")
        sid = lax.axis_index("s")
        wid = cid * NS + sid

        for r in range(zb):
            for v in range(RW // 16):
                zbuf[r, pl.ds(v * 16, 16)] = jnp.zeros((16,), jnp.float32)
        row0 = sid * rps

        def zloop(i, carry):
            pltpu.sync_copy(zbuf, accum.at[pl.ds(row0 + i * zb, zb)])
            return carry

        lax.fori_loop(0, rps // zb, zloop, 0)
        plsc.subcore_barrier()

        base_e = wid * (CPW * C)

        def issue_gather(k, g):
            b = base_e + g * C
            pltpu.sync_copy(src_hbm.at[pl.ds(b, C)], idx_s.at[k])
            pltpu.sync_copy(dst_hbm.at[pl.ds(b, C)], idx_d.at[k])
            pltpu.async_copy(s_hbm.at[idx_s.at[k]], rows_s.at[k], gsem[k])
            pltpu.async_copy(t_hbm.at[idx_d.at[k]], rows_d.at[k], gsem[k])

        def wait_gather(k):
            pltpu.make_async_copy(
                s_hbm.at[idx_s.at[k]], rows_s.at[k], gsem[k]).wait()
            pltpu.make_async_copy(
                t_hbm.at[idx_d.at[k]], rows_d.at[k], gsem[k]).wait()

        def issue_scatter(k):
            pltpu.async_copy(wbuf.at[k], accum.at[idx_d.at[k]], wsem[k],
                             add=True)

        def wait_scatter(k):
            pltpu.make_async_copy(
                wbuf.at[k], accum.at[idx_d.at[k]], wsem[k]).wait()

        def compute(k):
            lane_lo = lax.iota(jnp.int32, 16) < 8

            @plsc.parallel_loop(0, C, 1, unroll=8)
            def edge(e):
                a = rows_s[k, e, pl.ds(D // 2, 16)] + rows_d[k, e, :]
                s = jnp.exp(jnp.where(a >= 0.0, a, 0.2 * a))
                wbuf[k, e, pl.ds(D, 16)] = s
                for g4 in range(4):
                    u = lax.bitcast_convert_type(
                        rows_s[k, e, pl.ds(16 * g4, 16)], jnp.int32)
                    he = lax.bitcast_convert_type(u << 16, jnp.float32)
                    ho = lax.bitcast_convert_type(
                        u & jnp.int32(-65536), jnp.float32)
                    if nheads == 1:
                        sg = s
                    else:
                        s_lo = lax.gather(
                            s, jnp.full((16, 1), 2 * g4, jnp.int32),
                            _GATHER_DNUMS, (1,),
                            mode=lax.GatherScatterMode.PROMISE_IN_BOUNDS)
                        s_hi = lax.gather(
                            s, jnp.full((16, 1), 2 * g4 + 1, jnp.int32),
                            _GATHER_DNUMS, (1,),
                            mode=lax.GatherScatterMode.PROMISE_IN_BOUNDS)
                        sg = jnp.where(lane_lo, s_lo, s_hi)
                    wbuf[k, e, pl.ds(32 * g4, 16)] = he * sg
                    wbuf[k, e, pl.ds(32 * g4 + 16, 16)] = ho * sg

        issue_gather(0, 0)
        issue_gather(1, 1)

        def trip(t, carry):
            g = t * NB
            for k in range(NB):
                gk = g + k
                wait_gather(k)
                compute(k)
                issue_scatter(k)
                kp = (k + 2) % NB
                if k == 0:
                    @pl.when(t > 0)
                    def _():
                        wait_scatter(kp)
                else:
                    wait_scatter(kp)

                @pl.when(gk + 2 < CPW)
                def _():
                    issue_gather(kp, gk + 2)
            return carry

        lax.fori_loop(0, CPW // NB, trip, 0)
        wait_scatter((CPW - 1) % NB)
        plsc.subcore_barrier()

        def oloop(i, carry):
            r = row0 + i * ob
            pltpu.sync_copy(accum.at[pl.ds(r, ob)],
                            out_hbm.at[cid, pl.ds(r, ob)])
            return carry

        lax.fori_loop(0, rps // ob, oloop, 0)

    return edge_kernel


_edge8 = _make_edge_kernel(H1)
_edge1 = _make_edge_kernel(1)


def kernel(x, edge_index, W1, att_src1, att_dst1, b1, W2, att_src2,
           att_dst2, b2):
    x = x.astype(jnp.float32)
    xp = jnp.zeros((NPAD, D), jnp.float32).at[:N].set(x)
    ei = edge_index.astype(jnp.int32)
    loop = jnp.arange(N, dtype=jnp.int32)
    pad = jnp.full((EPAD - ET,), NPAD - 1, jnp.int32)
    src = jnp.concatenate([ei[0], loop, pad])
    dst = jnp.concatenate([ei[1], loop, pad])

    eye8 = jnp.eye(H1, dtype=jnp.float32)
    a_s1 = (att_src1[:, :, None] * eye8[:, None, :]).reshape(D, H1)
    a_d1 = (att_dst1[:, :, None] * eye8[:, None, :]).reshape(D, H1)
    rh = (eye8[:, :, None] * jnp.ones((1, 1, O1), jnp.float32)).reshape(H1, D)
    emat = jnp.asarray(_E_NP)
    omat = jnp.asarray(_O_NP)
    pmat = jnp.asarray(_P_NP)

    s1, t1 = _prep1(xp, W1, a_s1, a_d1, emat, omat)
    acc1 = _edge8(s1, t1, src, dst)
    s2, t2 = _mid(acc1, b1.reshape(1, D), W2, rh, pmat, emat, omat,
                  att_src2.reshape(1, D), att_dst2.reshape(1, D))
    acc2 = _edge1(s2, t2, src, dst)
    out = _fin(acc2, b2.reshape(1, D), pmat)
    return out[:N]
